# Initial kernel scaffold; baseline (speedup 1.0000x reference)
import functools

import jax
import jax.numpy as jnp
from jax import lax
from jax.experimental import pallas as pl
from jax.experimental.pallas import tpu as pltpu
from jax.experimental.pallas import tpu_sc as plsc

NC = 2
NS = 16
NW = NC * NS
L = 16


def _edge_kernel(feats, src, dst, *, n_nodes, n_edges, d_feat):
    ew = n_edges // NW
    C = 80
    nchunk = ew // C
    nseg = d_feat // L
    mesh = plsc.VectorSubcoreMesh(core_axis_name="c", subcore_axis_name="s")

    @functools.partial(
        pl.kernel,
        out_type=(
            jax.ShapeDtypeStruct((n_edges,), jnp.float32),
            jax.ShapeDtypeStruct((NC, n_nodes), jnp.float32),
        ),
        mesh=mesh,
        scratch_types=[
            pltpu.VMEM((C,), jnp.int32),
            pltpu.VMEM((C,), jnp.int32),
            pltpu.VMEM((C, d_feat), jnp.float32),
            pltpu.VMEM((C, d_feat), jnp.float32),
            pltpu.VMEM((C,), jnp.float32),
            pltpu.VMEM((C,), jnp.float32),
            pltpu.VMEM((n_nodes,), jnp.float32),
            pltpu.VMEM_SHARED((n_nodes,), jnp.float32),
            pltpu.SemaphoreType.DMA,
            pltpu.SemaphoreType.DMA,
        ],
    )
    def body(feats_hbm, src_hbm, dst_hbm, t_hbm, s2_hbm,
             idx_s, idx_d, a, b, dbuf, tbuf, zbuf, s_sh, sem_a, sem_b):
        cid = lax.axis_index("c")
        sid = lax.axis_index("s")
        wid = sid * NC + cid

        @pl.when(sid == 0)
        def _():
            @pl.loop(0, n_nodes, step=L)
            def _(i):
                zbuf[pl.ds(i, L)] = jnp.zeros((L,), jnp.float32)
            pltpu.sync_copy(zbuf, s_sh)

        plsc.subcore_barrier()

        base0 = wid * ew

        @pl.loop(0, nchunk)
        def _(k):
            base = base0 + k * C
            pltpu.sync_copy(src_hbm.at[pl.ds(base, C)], idx_s)
            pltpu.sync_copy(dst_hbm.at[pl.ds(base, C)], idx_d)
            ca = pltpu.async_copy(feats_hbm.at[idx_s], a, sem_a)
            cb = pltpu.async_copy(feats_hbm.at[idx_d], b, sem_b)
            ca.wait()
            cb.wait()

            @pl.loop(0, C)
            def _(i):
                acc = jnp.zeros((L,), jnp.float32)
                for j in range(nseg):
                    av = a[i, pl.ds(j * L, L)]
                    bv = b[i, pl.ds(j * L, L)]
                    acc = acc + jnp.abs(av - bv)
                dbuf[i] = jnp.sum(acc)

            @pl.loop(0, C, step=L)
            def _(i):
                dv = dbuf[pl.ds(i, L)]
                tv = jnp.exp(jnp.exp(dv * (-0.01)))
                tbuf[pl.ds(i, L)] = tv

            pltpu.sync_copy(tbuf, t_hbm.at[pl.ds(base, C)])
            pltpu.sync_copy(tbuf, s_sh.at[idx_d], add=True)

        plsc.subcore_barrier()

        @pl.when(sid == 0)
        def _():
            pltpu.sync_copy(s_sh, s2_hbm.at[cid])

    return body(feats, src, dst)


def _norm_kernel(t, dst, s2, *, n_nodes, n_edges):
    ew = n_edges // NW
    C = 1000
    nchunk = ew // C
    mesh = plsc.VectorSubcoreMesh(core_axis_name="c", subcore_axis_name="s")

    @functools.partial(
        pl.kernel,
        out_type=jax.ShapeDtypeStruct((n_edges,), jnp.float32),
        mesh=mesh,
        scratch_types=[
            pltpu.VMEM((n_nodes,), jnp.float32),
            pltpu.VMEM((n_nodes,), jnp.float32),
            pltpu.VMEM((C,), jnp.int32),
            pltpu.VMEM((C,), jnp.float32),
            pltpu.VMEM((C,), jnp.float32),
        ],
    )
    def body(t_hbm, dst_hbm, s2_hbm, out_hbm, s_v, s_v2, idx, tv, ov):
        cid = lax.axis_index("c")
        sid = lax.axis_index("s")
        wid = sid * NC + cid

        pltpu.sync_copy(s2_hbm.at[0], s_v)
        pltpu.sync_copy(s2_hbm.at[1], s_v2)

        @pl.loop(0, n_nodes, step=L)
        def _(i):
            s_v[pl.ds(i, L)] = s_v[pl.ds(i, L)] + s_v2[pl.ds(i, L)]

        base0 = wid * ew

        @pl.loop(0, nchunk)
        def _(k):
            base = base0 + k * C
            pltpu.sync_copy(dst_hbm.at[pl.ds(base, C)], idx)
            pltpu.sync_copy(t_hbm.at[pl.ds(base, C)], tv)

            @pl.loop(0, C, step=L)
            def _(i):
                iv = idx[pl.ds(i, L)]
                g = plsc.load_gather(s_v, [iv])
                ov[pl.ds(i, L)] = tv[pl.ds(i, L)] / g

            pltpu.sync_copy(ov, out_hbm.at[pl.ds(base, C)])

    return body(t, dst, s2)


def kernel(feats, edge_index):
    n_nodes, d_feat = feats.shape
    n_edges = edge_index.shape[1]
    src = edge_index[0].astype(jnp.int32)
    dst = edge_index[1].astype(jnp.int32)
    t, s2 = _edge_kernel(feats, src, dst,
                         n_nodes=n_nodes, n_edges=n_edges, d_feat=d_feat)
    out = _norm_kernel(t, dst, s2, n_nodes=n_nodes, n_edges=n_edges)
    return out.reshape(n_edges, 1)


# speedup vs baseline: 9.6271x; 9.6271x over previous
import dataclasses
import functools

import jax
import jax.numpy as jnp
from jax import lax
from jax.experimental import pallas as pl
from jax.experimental.pallas import tpu as pltpu
from jax.experimental.pallas import tpu_sc as plsc

NC = 2
NS = 16
NW = NC * NS
L = 16


def _sc_compiler_params():
    cp = pltpu.CompilerParams()
    if "needs_layout_passes" in pltpu.CompilerParams.__dataclass_fields__:
        cp = dataclasses.replace(cp, needs_layout_passes=False)
    return cp


def _edge_kernel(feats, src, dst, *, n_nodes, n_edges, d_feat):
    ew = n_edges // NW
    C = 80
    nchunk = ew // C
    nseg = d_feat // L
    mesh = plsc.VectorSubcoreMesh(core_axis_name="c", subcore_axis_name="s")

    @functools.partial(
        pl.kernel,
        out_type=(
            jax.ShapeDtypeStruct((n_edges,), jnp.float32),
            jax.ShapeDtypeStruct((NW, n_nodes), jnp.float32),
        ),
        mesh=mesh,
        scratch_types=[
            pltpu.VMEM((C,), jnp.int32),
            pltpu.VMEM((C,), jnp.int32),
            pltpu.VMEM((C, d_feat), jnp.float32),
            pltpu.VMEM((C, d_feat), jnp.float32),
            pltpu.VMEM((C,), jnp.float32),
            pltpu.VMEM((C,), jnp.float32),
            pltpu.VMEM((n_nodes,), jnp.float32),
            pltpu.SemaphoreType.DMA,
            pltpu.SemaphoreType.DMA,
        ],
        compiler_params=_sc_compiler_params(),
    )
    def body(feats_hbm, src_hbm, dst_hbm, t_hbm, sp_hbm,
             idx_s, idx_d, a, b, dbuf, tbuf, s_local, sem_a, sem_b):
        cid = lax.axis_index("c")
        sid = lax.axis_index("s")
        wid = sid * NC + cid

        @pl.loop(0, n_nodes, step=L)
        def _(i):
            s_local[pl.ds(i, L)] = jnp.zeros((L,), jnp.float32)

        base0 = wid * ew
        lane = lax.iota(jnp.int32, L)
        last_lane = lane == (L - 1)

        @pl.loop(0, nchunk)
        def _(k):
            base = base0 + k * C
            pltpu.sync_copy(src_hbm.at[pl.ds(base, C)], idx_s)
            pltpu.sync_copy(dst_hbm.at[pl.ds(base, C)], idx_d)
            ca = pltpu.async_copy(feats_hbm.at[idx_s], a, sem_a)
            cb = pltpu.async_copy(feats_hbm.at[idx_d], b, sem_b)
            ca.wait()
            cb.wait()

            @pl.loop(0, C)
            def _(i):
                acc = jnp.zeros((L,), jnp.float32)
                for j in range(nseg):
                    av = a[i, pl.ds(j * L, L)]
                    bv = b[i, pl.ds(j * L, L)]
                    acc = acc + jnp.abs(av - bv)
                csum = plsc.cumsum(acc)
                plsc.store_scatter(dbuf, [jnp.full((L,), i, jnp.int32)],
                                   csum, mask=last_lane)

            @pl.loop(0, C, step=L)
            def _(i):
                dv = dbuf[pl.ds(i, L)]
                tv = jnp.exp(jnp.exp(dv * (-0.01)))
                tbuf[pl.ds(i, L)] = tv
                iv = idx_d[pl.ds(i, L)]
                plsc.addupdate_scatter(s_local, [iv], tv)

            pltpu.sync_copy(tbuf, t_hbm.at[pl.ds(base, C)])

        pltpu.sync_copy(s_local, sp_hbm.at[wid])

    return body(feats, src, dst)


def _combine_kernel(sp, *, n_nodes):
    def body(sp_ref, r_ref):
        r_ref[...] = 1.0 / jnp.sum(sp_ref[...], axis=0, keepdims=True)

    return pl.pallas_call(
        body,
        out_shape=jax.ShapeDtypeStruct((1, n_nodes), jnp.float32),
    )(sp)


def _norm_kernel(t, dst, r, *, n_nodes, n_edges):
    ew = n_edges // NW
    C = 80
    nchunk = ew // C
    mesh = plsc.VectorSubcoreMesh(core_axis_name="c", subcore_axis_name="s")

    @functools.partial(
        pl.kernel,
        out_type=jax.ShapeDtypeStruct((n_edges,), jnp.float32),
        mesh=mesh,
        scratch_types=[
            pltpu.VMEM((C,), jnp.int32),
            pltpu.VMEM((C,), jnp.float32),
            pltpu.VMEM((C,), jnp.float32),
            pltpu.VMEM((C,), jnp.float32),
            pltpu.SemaphoreType.DMA,
        ],
        compiler_params=_sc_compiler_params(),
    )
    def body(t_hbm, dst_hbm, r_hbm, out_hbm, idx, tv, rv, ov, sem):
        cid = lax.axis_index("c")
        sid = lax.axis_index("s")
        wid = sid * NC + cid
        base0 = wid * ew

        @pl.loop(0, nchunk)
        def _(k):
            base = base0 + k * C
            pltpu.sync_copy(dst_hbm.at[pl.ds(base, C)], idx)
            cg = pltpu.async_copy(r_hbm.at[idx], rv, sem)
            pltpu.sync_copy(t_hbm.at[pl.ds(base, C)], tv)
            cg.wait()

            @pl.loop(0, C, step=L)
            def _(i):
                ov[pl.ds(i, L)] = tv[pl.ds(i, L)] * rv[pl.ds(i, L)]

            pltpu.sync_copy(ov, out_hbm.at[pl.ds(base, C)])

    return body(t, dst, r)


def kernel(feats, edge_index):
    n_nodes, d_feat = feats.shape
    n_edges = edge_index.shape[1]
    src = edge_index[0].astype(jnp.int32)
    dst = edge_index[1].astype(jnp.int32)
    t, sp = _edge_kernel(feats, src, dst,
                         n_nodes=n_nodes, n_edges=n_edges, d_feat=d_feat)
    r = _combine_kernel(sp, n_nodes=n_nodes).reshape(n_nodes)
    out = _norm_kernel(t, dst, r, n_nodes=n_nodes, n_edges=n_edges)
    return out.reshape(n_edges, 1)


# speedup vs baseline: 21.7512x; 2.2594x over previous
import dataclasses
import functools

import jax
import jax.numpy as jnp
from jax import lax
from jax.experimental import pallas as pl
from jax.experimental.pallas import tpu as pltpu
from jax.experimental.pallas import tpu_sc as plsc

NC = 2
NS = 16
NW = NC * NS
L = 16


def _sc_compiler_params():
    cp = pltpu.CompilerParams()
    if "needs_layout_passes" in pltpu.CompilerParams.__dataclass_fields__:
        cp = dataclasses.replace(cp, needs_layout_passes=False)
    return cp


def _edge_kernel(feats, src, dst, *, n_nodes, n_edges, d_feat):
    ew = n_edges // NW
    C = 80
    nchunk = ew // C
    nseg = d_feat // L
    mesh = plsc.VectorSubcoreMesh(core_axis_name="c", subcore_axis_name="s")

    @functools.partial(
        pl.kernel,
        out_type=(
            jax.ShapeDtypeStruct((n_edges,), jnp.float32),
            jax.ShapeDtypeStruct((NW, n_nodes), jnp.float32),
        ),
        mesh=mesh,
        scratch_types=[
            pltpu.VMEM((ew,), jnp.int32),
            pltpu.VMEM((ew,), jnp.int32),
            pltpu.VMEM((C, d_feat), jnp.float32),
            pltpu.VMEM((C, d_feat), jnp.float32),
            pltpu.VMEM((C, d_feat), jnp.float32),
            pltpu.VMEM((C, d_feat), jnp.float32),
            pltpu.VMEM((C,), jnp.float32),
            pltpu.VMEM((ew,), jnp.float32),
            pltpu.VMEM((n_nodes,), jnp.float32),
            pltpu.SemaphoreType.DMA,
            pltpu.SemaphoreType.DMA,
            pltpu.SemaphoreType.DMA,
            pltpu.SemaphoreType.DMA,
        ],
        compiler_params=_sc_compiler_params(),
    )
    def body(feats_hbm, src_hbm, dst_hbm, t_hbm, sp_hbm,
             idxs_all, idxd_all, a0, b0, a1, b1, dbuf, t_all, s_local,
             sa0, sb0, sa1, sb1):
        cid = lax.axis_index("c")
        sid = lax.axis_index("s")
        wid = sid * NC + cid
        base0 = wid * ew

        pltpu.sync_copy(src_hbm.at[pl.ds(base0, ew)], idxs_all)
        pltpu.sync_copy(dst_hbm.at[pl.ds(base0, ew)], idxd_all)

        @pl.loop(0, n_nodes, step=L)
        def _(i):
            s_local[pl.ds(i, L)] = jnp.zeros((L,), jnp.float32)

        lane = lax.iota(jnp.int32, L)
        last_lane = lane == (L - 1)

        def fire(k, a_buf, b_buf, sa, sb):
            pltpu.async_copy(feats_hbm.at[idxs_all.at[pl.ds(k * C, C)]],
                             a_buf, sa)
            pltpu.async_copy(feats_hbm.at[idxd_all.at[pl.ds(k * C, C)]],
                             b_buf, sb)

        def wait(k, a_buf, b_buf, sa, sb):
            pltpu.make_async_copy(feats_hbm.at[idxs_all.at[pl.ds(k * C, C)]],
                                  a_buf, sa).wait()
            pltpu.make_async_copy(feats_hbm.at[idxd_all.at[pl.ds(k * C, C)]],
                                  b_buf, sb).wait()

        def compute(k, a_buf, b_buf):
            @pl.loop(0, C)
            def _(i):
                acc = jnp.zeros((L,), jnp.float32)
                for j in range(nseg):
                    av = a_buf[i, pl.ds(j * L, L)]
                    bv = b_buf[i, pl.ds(j * L, L)]
                    acc = acc + jnp.abs(av - bv)
                csum = plsc.cumsum(acc)
                plsc.store_scatter(dbuf, [jnp.full((L,), i, jnp.int32)],
                                   csum, mask=last_lane)

            @pl.loop(0, C, step=L)
            def _(i):
                dv = dbuf[pl.ds(i, L)]
                tv = jnp.exp(jnp.exp(dv * (-0.01)))
                t_all[pl.ds(k * C + i, L)] = tv
                iv = idxd_all[pl.ds(k * C + i, L)]
                plsc.addupdate_scatter(s_local, [iv], tv)

        fire(0, a0, b0, sa0, sb0)

        @pl.loop(0, nchunk - 1, step=2)
        def _(k):
            fire(k + 1, a1, b1, sa1, sb1)
            wait(k, a0, b0, sa0, sb0)
            compute(k, a0, b0)
            fire(k + 2, a0, b0, sa0, sb0)
            wait(k + 1, a1, b1, sa1, sb1)
            compute(k + 1, a1, b1)

        klast = nchunk - 1
        wait(klast, a0, b0, sa0, sb0)
        compute(klast, a0, b0)

        pltpu.sync_copy(t_all, t_hbm.at[pl.ds(base0, ew)])
        pltpu.sync_copy(s_local, sp_hbm.at[wid])

    return body(feats, src, dst)


def _combine_kernel(sp, *, n_nodes):
    def body(sp_ref, r_ref):
        r_ref[...] = 1.0 / jnp.sum(sp_ref[...], axis=0, keepdims=True)

    return pl.pallas_call(
        body,
        out_shape=jax.ShapeDtypeStruct((1, n_nodes), jnp.float32),
    )(sp)


def _norm_kernel(t, dst, r, *, n_nodes, n_edges):
    ew = n_edges // NW
    G = 80
    ngroups = ew // G
    mesh = plsc.VectorSubcoreMesh(core_axis_name="c", subcore_axis_name="s")

    @functools.partial(
        pl.kernel,
        out_type=jax.ShapeDtypeStruct((n_edges,), jnp.float32),
        mesh=mesh,
        scratch_types=[
            pltpu.VMEM((ew,), jnp.int32),
            pltpu.VMEM((ew,), jnp.float32),
            pltpu.VMEM((ew,), jnp.float32),
            pltpu.SemaphoreType.DMA,
        ],
        compiler_params=_sc_compiler_params(),
    )
    def body(t_hbm, dst_hbm, r_hbm, out_hbm, idx_all, t_all, rv_all, sem):
        cid = lax.axis_index("c")
        sid = lax.axis_index("s")
        wid = sid * NC + cid
        base0 = wid * ew

        pltpu.sync_copy(dst_hbm.at[pl.ds(base0, ew)], idx_all)

        @pl.loop(0, ngroups)
        def _(j):
            pltpu.async_copy(r_hbm.at[idx_all.at[pl.ds(j * G, G)]],
                             rv_all.at[pl.ds(j * G, G)], sem)

        pltpu.sync_copy(t_hbm.at[pl.ds(base0, ew)], t_all)

        @pl.loop(0, ngroups)
        def _(j):
            pltpu.make_async_copy(r_hbm.at[idx_all.at[pl.ds(j * G, G)]],
                                  rv_all.at[pl.ds(j * G, G)], sem).wait()

        @pl.loop(0, ew, step=L)
        def _(i):
            t_all[pl.ds(i, L)] = t_all[pl.ds(i, L)] * rv_all[pl.ds(i, L)]

        pltpu.sync_copy(t_all, out_hbm.at[pl.ds(base0, ew)])

    return body(t, dst, r)


def kernel(feats, edge_index):
    n_nodes, d_feat = feats.shape
    n_edges = edge_index.shape[1]
    src = edge_index[0].astype(jnp.int32)
    dst = edge_index[1].astype(jnp.int32)
    t, sp = _edge_kernel(feats, src, dst,
                         n_nodes=n_nodes, n_edges=n_edges, d_feat=d_feat)
    r = _combine_kernel(sp, n_nodes=n_nodes).reshape(n_nodes)
    out = _norm_kernel(t, dst, r, n_nodes=n_nodes, n_edges=n_edges)
    return out.reshape(n_edges, 1)


# speedup vs baseline: 28.5141x; 1.3109x over previous
import dataclasses
import functools

import jax
import jax.numpy as jnp
from jax import lax
from jax.experimental import pallas as pl
from jax.experimental.pallas import tpu as pltpu
from jax.experimental.pallas import tpu_sc as plsc

NC = 2
NS = 16
NW = NC * NS
L = 16


def _sc_compiler_params():
    cp = pltpu.CompilerParams()
    if "needs_layout_passes" in pltpu.CompilerParams.__dataclass_fields__:
        cp = dataclasses.replace(cp, needs_layout_passes=False)
    return cp


def _edge_kernel(feats, src, dst, *, n_nodes, n_edges, d_feat):
    ew = n_edges // NW
    C = 80
    nchunk = ew // C
    nseg = d_feat // L
    mesh = plsc.VectorSubcoreMesh(core_axis_name="c", subcore_axis_name="s")

    @functools.partial(
        pl.kernel,
        out_type=(
            jax.ShapeDtypeStruct((n_edges,), jnp.float32),
            jax.ShapeDtypeStruct((NW, n_nodes), jnp.float32),
        ),
        mesh=mesh,
        scratch_types=[
            pltpu.VMEM((ew,), jnp.int32),
            pltpu.VMEM((ew,), jnp.int32),
            pltpu.VMEM((C, d_feat), jnp.float32),
            pltpu.VMEM((C, d_feat), jnp.float32),
            pltpu.VMEM((C, d_feat), jnp.float32),
            pltpu.VMEM((C, d_feat), jnp.float32),
            pltpu.VMEM((C,), jnp.float32),
            pltpu.VMEM((ew,), jnp.float32),
            pltpu.VMEM((n_nodes,), jnp.float32),
            pltpu.SemaphoreType.DMA,
            pltpu.SemaphoreType.DMA,
            pltpu.SemaphoreType.DMA,
            pltpu.SemaphoreType.DMA,
        ],
        compiler_params=_sc_compiler_params(),
    )
    def body(feats_hbm, src_hbm, dst_hbm, t_hbm, sp_hbm,
             idxs_all, idxd_all, a0, b0, a1, b1, dbuf, t_all, s_local,
             sa0, sb0, sa1, sb1):
        cid = lax.axis_index("c")
        sid = lax.axis_index("s")
        wid = sid * NC + cid
        base0 = wid * ew

        pltpu.sync_copy(src_hbm.at[pl.ds(base0, ew)], idxs_all)
        pltpu.sync_copy(dst_hbm.at[pl.ds(base0, ew)], idxd_all)

        @pl.loop(0, n_nodes, step=L)
        def _(i):
            s_local[pl.ds(i, L)] = jnp.zeros((L,), jnp.float32)

        lane = lax.iota(jnp.int32, L)
        last_lane = lane == (L - 1)

        def fire(k, a_buf, b_buf, sa, sb):
            pltpu.async_copy(feats_hbm.at[idxs_all.at[pl.ds(k * C, C)]],
                             a_buf, sa)
            pltpu.async_copy(feats_hbm.at[idxd_all.at[pl.ds(k * C, C)]],
                             b_buf, sb)

        def wait(k, a_buf, b_buf, sa, sb):
            pltpu.make_async_copy(feats_hbm.at[idxs_all.at[pl.ds(k * C, C)]],
                                  a_buf, sa).wait()
            pltpu.make_async_copy(feats_hbm.at[idxd_all.at[pl.ds(k * C, C)]],
                                  b_buf, sb).wait()

        def compute(k, a_buf, b_buf):
            @plsc.parallel_loop(0, C, unroll=4)
            def _(i):
                accs = []
                for j in range(4):
                    av = a_buf[i, pl.ds(j * L, L)]
                    bv = b_buf[i, pl.ds(j * L, L)]
                    accs.append(jnp.abs(av - bv))
                for j in range(4, nseg):
                    av = a_buf[i, pl.ds(j * L, L)]
                    bv = b_buf[i, pl.ds(j * L, L)]
                    accs[j - 4] = accs[j - 4] + jnp.abs(av - bv)
                acc = (accs[0] + accs[1]) + (accs[2] + accs[3])
                csum = plsc.cumsum(acc)
                plsc.store_scatter(dbuf, [jnp.full((L,), i, jnp.int32)],
                                   csum, mask=last_lane)

            @pl.loop(0, C, step=L)
            def _(i):
                dv = dbuf[pl.ds(i, L)]
                tv = jnp.exp(jnp.exp(dv * (-0.01)))
                t_all[pl.ds(k * C + i, L)] = tv
                iv = idxd_all[pl.ds(k * C + i, L)]
                plsc.addupdate_scatter(s_local, [iv], tv)

        fire(0, a0, b0, sa0, sb0)

        @pl.loop(0, nchunk - 1, step=2)
        def _(k):
            fire(k + 1, a1, b1, sa1, sb1)
            wait(k, a0, b0, sa0, sb0)
            compute(k, a0, b0)
            fire(k + 2, a0, b0, sa0, sb0)
            wait(k + 1, a1, b1, sa1, sb1)
            compute(k + 1, a1, b1)

        klast = nchunk - 1
        wait(klast, a0, b0, sa0, sb0)
        compute(klast, a0, b0)

        pltpu.sync_copy(t_all, t_hbm.at[pl.ds(base0, ew)])
        pltpu.sync_copy(s_local, sp_hbm.at[wid])

    return body(feats, src, dst)


def _combine_kernel(sp, *, n_nodes):
    def body(sp_ref, r_ref):
        r_ref[...] = 1.0 / jnp.sum(sp_ref[...], axis=0, keepdims=True)

    return pl.pallas_call(
        body,
        out_shape=jax.ShapeDtypeStruct((1, n_nodes), jnp.float32),
    )(sp)


def _norm_kernel(t, dst, r, *, n_nodes, n_edges):
    ew = n_edges // NW
    G = 80
    ngroups = ew // G
    mesh = plsc.VectorSubcoreMesh(core_axis_name="c", subcore_axis_name="s")

    @functools.partial(
        pl.kernel,
        out_type=jax.ShapeDtypeStruct((n_edges,), jnp.float32),
        mesh=mesh,
        scratch_types=[
            pltpu.VMEM((ew,), jnp.int32),
            pltpu.VMEM((ew,), jnp.float32),
            pltpu.VMEM((ew,), jnp.float32),
            pltpu.SemaphoreType.DMA,
        ],
        compiler_params=_sc_compiler_params(),
    )
    def body(t_hbm, dst_hbm, r_hbm, out_hbm, idx_all, t_all, rv_all, sem):
        cid = lax.axis_index("c")
        sid = lax.axis_index("s")
        wid = sid * NC + cid
        base0 = wid * ew

        pltpu.sync_copy(dst_hbm.at[pl.ds(base0, ew)], idx_all)

        @pl.loop(0, ngroups)
        def _(j):
            pltpu.async_copy(r_hbm.at[idx_all.at[pl.ds(j * G, G)]],
                             rv_all.at[pl.ds(j * G, G)], sem)

        pltpu.sync_copy(t_hbm.at[pl.ds(base0, ew)], t_all)

        @pl.loop(0, ngroups)
        def _(j):
            pltpu.make_async_copy(r_hbm.at[idx_all.at[pl.ds(j * G, G)]],
                                  rv_all.at[pl.ds(j * G, G)], sem).wait()

        @pl.loop(0, ew, step=L)
        def _(i):
            t_all[pl.ds(i, L)] = t_all[pl.ds(i, L)] * rv_all[pl.ds(i, L)]

        pltpu.sync_copy(t_all, out_hbm.at[pl.ds(base0, ew)])

    return body(t, dst, r)


def kernel(feats, edge_index):
    n_nodes, d_feat = feats.shape
    n_edges = edge_index.shape[1]
    src = edge_index[0].astype(jnp.int32)
    dst = edge_index[1].astype(jnp.int32)
    t, sp = _edge_kernel(feats, src, dst,
                         n_nodes=n_nodes, n_edges=n_edges, d_feat=d_feat)
    r = _combine_kernel(sp, n_nodes=n_nodes).reshape(n_nodes)
    out = _norm_kernel(t, dst, r, n_nodes=n_nodes, n_edges=n_edges)
    return out.reshape(n_edges, 1)


# speedup vs baseline: 30.5822x; 1.0725x over previous
import dataclasses
import functools

import jax
import jax.numpy as jnp
from jax import lax
from jax.experimental import pallas as pl
from jax.experimental.pallas import tpu as pltpu
from jax.experimental.pallas import tpu_sc as plsc

NC = 2
NS = 16
NW = NC * NS
L = 16


def _sc_compiler_params(tc_tiling=None):
    cp = pltpu.CompilerParams()
    if "needs_layout_passes" in pltpu.CompilerParams.__dataclass_fields__:
        cp = dataclasses.replace(cp, needs_layout_passes=False)
    if tc_tiling is not None:
        cp = dataclasses.replace(cp, use_tc_tiling_on_sc=tc_tiling)
    return cp


def _edge_kernel(feats, src, dst, *, n_nodes, n_edges, d_feat):
    ew = n_edges // NW
    C = 80
    nchunk = ew // C
    nseg2 = d_feat // (2 * L)
    mesh = plsc.VectorSubcoreMesh(core_axis_name="c", subcore_axis_name="s")

    @functools.partial(
        pl.kernel,
        out_type=(
            jax.ShapeDtypeStruct((n_edges,), jnp.float32),
            jax.ShapeDtypeStruct((NW, n_nodes), jnp.float32),
        ),
        mesh=mesh,
        scratch_types=[
            pltpu.VMEM((ew,), jnp.int32),
            pltpu.VMEM((ew,), jnp.int32),
            pltpu.VMEM((C, d_feat // 2), jnp.int32),
            pltpu.VMEM((C, d_feat // 2), jnp.int32),
            pltpu.VMEM((C, d_feat // 2), jnp.int32),
            pltpu.VMEM((C, d_feat // 2), jnp.int32),
            pltpu.VMEM((C,), jnp.float32),
            pltpu.VMEM((ew,), jnp.float32),
            pltpu.VMEM((n_nodes,), jnp.float32),
            pltpu.SemaphoreType.DMA,
            pltpu.SemaphoreType.DMA,
            pltpu.SemaphoreType.DMA,
            pltpu.SemaphoreType.DMA,
        ],
        compiler_params=_sc_compiler_params(tc_tiling=False),
    )
    def body(feats_hbm, src_hbm, dst_hbm, t_hbm, sp_hbm,
             idxs_all, idxd_all, a0, b0, a1, b1, dbuf, t_all, s_local,
             sa0, sb0, sa1, sb1):
        cid = lax.axis_index("c")
        sid = lax.axis_index("s")
        wid = sid * NC + cid
        base0 = wid * ew

        pltpu.sync_copy(src_hbm.at[pl.ds(base0, ew)], idxs_all)
        pltpu.sync_copy(dst_hbm.at[pl.ds(base0, ew)], idxd_all)

        @pl.loop(0, n_nodes, step=L)
        def _(i):
            s_local[pl.ds(i, L)] = jnp.zeros((L,), jnp.float32)

        lane = lax.iota(jnp.int32, L)
        last_lane = lane == (L - 1)

        def fire(k, a_buf, b_buf, sa, sb):
            pltpu.async_copy(feats_hbm.at[idxs_all.at[pl.ds(k * C, C)]],
                             a_buf, sa)
            pltpu.async_copy(feats_hbm.at[idxd_all.at[pl.ds(k * C, C)]],
                             b_buf, sb)

        def wait(k, a_buf, b_buf, sa, sb):
            pltpu.make_async_copy(feats_hbm.at[idxs_all.at[pl.ds(k * C, C)]],
                                  a_buf, sa).wait()
            pltpu.make_async_copy(feats_hbm.at[idxd_all.at[pl.ds(k * C, C)]],
                                  b_buf, sb).wait()

        def compute(k, a_buf, b_buf):
            @plsc.parallel_loop(0, C, unroll=4)
            def _(i):
                parts = []
                for j in range(nseg2):
                    av = plsc.bitcast(a_buf[i, pl.ds(j * L, L)], jnp.bfloat16)
                    bv = plsc.bitcast(b_buf[i, pl.ds(j * L, L)], jnp.bfloat16)
                    dj = jnp.abs(av - bv)
                    lo, hi = plsc.unpack(dj, format=plsc.PackFormat.INTERLEAVED,
                                         preferred_element_type=jnp.float32)
                    parts.append(lo + hi)
                acc = (parts[0] + parts[1]) + (parts[2] + parts[3])
                csum = plsc.cumsum(acc)
                plsc.store_scatter(dbuf, [jnp.full((L,), i, jnp.int32)],
                                   csum, mask=last_lane)

            @pl.loop(0, C, step=L)
            def _(i):
                dv = dbuf[pl.ds(i, L)]
                tv = jnp.exp(jnp.exp(dv * (-0.01)))
                t_all[pl.ds(k * C + i, L)] = tv
                iv = idxd_all[pl.ds(k * C + i, L)]
                plsc.addupdate_scatter(s_local, [iv], tv)

        fire(0, a0, b0, sa0, sb0)

        @pl.loop(0, nchunk - 1, step=2)
        def _(k):
            fire(k + 1, a1, b1, sa1, sb1)
            wait(k, a0, b0, sa0, sb0)
            compute(k, a0, b0)
            fire(k + 2, a0, b0, sa0, sb0)
            wait(k + 1, a1, b1, sa1, sb1)
            compute(k + 1, a1, b1)

        klast = nchunk - 1
        wait(klast, a0, b0, sa0, sb0)
        compute(klast, a0, b0)

        pltpu.sync_copy(t_all, t_hbm.at[pl.ds(base0, ew)])
        pltpu.sync_copy(s_local, sp_hbm.at[wid])

    return body(feats, src, dst)


def _combine_kernel(sp, *, n_nodes):
    def body(sp_ref, r_ref):
        r_ref[...] = 1.0 / jnp.sum(sp_ref[...], axis=0, keepdims=True)

    return pl.pallas_call(
        body,
        out_shape=jax.ShapeDtypeStruct((1, n_nodes), jnp.float32),
    )(sp)


def _norm_kernel(t, dst, r, *, n_nodes, n_edges):
    ew = n_edges // NW
    G = 80
    ngroups = ew // G
    mesh = plsc.VectorSubcoreMesh(core_axis_name="c", subcore_axis_name="s")

    @functools.partial(
        pl.kernel,
        out_type=jax.ShapeDtypeStruct((n_edges,), jnp.float32),
        mesh=mesh,
        scratch_types=[
            pltpu.VMEM((ew,), jnp.int32),
            pltpu.VMEM((ew,), jnp.float32),
            pltpu.VMEM((ew,), jnp.float32),
            pltpu.SemaphoreType.DMA,
        ],
        compiler_params=_sc_compiler_params(),
    )
    def body(t_hbm, dst_hbm, r_hbm, out_hbm, idx_all, t_all, rv_all, sem):
        cid = lax.axis_index("c")
        sid = lax.axis_index("s")
        wid = sid * NC + cid
        base0 = wid * ew

        pltpu.sync_copy(dst_hbm.at[pl.ds(base0, ew)], idx_all)

        @pl.loop(0, ngroups)
        def _(j):
            pltpu.async_copy(r_hbm.at[idx_all.at[pl.ds(j * G, G)]],
                             rv_all.at[pl.ds(j * G, G)], sem)

        pltpu.sync_copy(t_hbm.at[pl.ds(base0, ew)], t_all)

        @pl.loop(0, ngroups)
        def _(j):
            pltpu.make_async_copy(r_hbm.at[idx_all.at[pl.ds(j * G, G)]],
                                  rv_all.at[pl.ds(j * G, G)], sem).wait()

        @pl.loop(0, ew, step=L)
        def _(i):
            t_all[pl.ds(i, L)] = t_all[pl.ds(i, L)] * rv_all[pl.ds(i, L)]

        pltpu.sync_copy(t_all, out_hbm.at[pl.ds(base0, ew)])

    return body(t, dst, r)


def kernel(feats, edge_index):
    n_nodes, d_feat = feats.shape
    n_edges = edge_index.shape[1]
    src = edge_index[0].astype(jnp.int32)
    dst = edge_index[1].astype(jnp.int32)
    feats = jax.lax.bitcast_convert_type(
        feats.astype(jnp.bfloat16).reshape(n_nodes, d_feat // 2, 2),
        jnp.int32)
    t, sp = _edge_kernel(feats, src, dst,
                         n_nodes=n_nodes, n_edges=n_edges, d_feat=d_feat)
    r = _combine_kernel(sp, n_nodes=n_nodes).reshape(n_nodes)
    out = _norm_kernel(t, dst, r, n_nodes=n_nodes, n_edges=n_edges)
    return out.reshape(n_edges, 1)
